# baseline (speedup 1.0000x reference)
import jax
import jax.numpy as jnp
from jax.experimental import pallas as pl
from jax.experimental.pallas import tpu as pltpu


BLK = 2048
POOL = 64
SEL = 8


def _body(x_ref, prompts_ref, keys_ref, out_ref, kn_ref, pm_ref):
    @pl.when(pl.program_id(0) == 0)
    def _init():
        keys = keys_ref[...]
        kn_ref[...] = keys * jax.lax.rsqrt(
            jnp.maximum(jnp.sum(keys * keys, axis=1, keepdims=True), 1e-24))
        pm_ref[...] = (jnp.mean(prompts_ref[...], axis=1)
                       * (1.0 / SEL)).astype(jnp.bfloat16)

    x = x_ref[...]
    sim = jax.lax.dot_general(
        x, kn_ref[...], (((1,), (1,)), ((), ())),
        preferred_element_type=jnp.float32)

    work = sim
    for _ in range(SEL - 1):
        m = jnp.max(work, axis=1, keepdims=True)
        work = jnp.where(work < m, work, -jnp.inf)
    t = jnp.max(work, axis=1, keepdims=True)
    sel = (sim >= t).astype(jnp.bfloat16)

    pf = jax.lax.dot_general(
        sel, pm_ref[...], (((1,), (0,)), ((), ())),
        preferred_element_type=jnp.float32)
    out_ref[...] = x + pf


@jax.jit
def kernel(x, prompts, keys):
    batch, dim = x.shape
    grid = (batch // BLK,)
    return pl.pallas_call(
        _body,
        grid=grid,
        in_specs=[
            pl.BlockSpec((BLK, dim), lambda i: (i, 0)),
            pl.BlockSpec(prompts.shape, lambda i: (0, 0, 0)),
            pl.BlockSpec(keys.shape, lambda i: (0, 0)),
        ],
        out_specs=pl.BlockSpec((BLK, dim), lambda i: (i, 0)),
        out_shape=jax.ShapeDtypeStruct((batch, dim), x.dtype),
        scratch_shapes=[
            pltpu.VMEM((POOL, dim), jnp.float32),
            pltpu.VMEM((POOL, dim), jnp.bfloat16),
        ],
        compiler_params=pltpu.CompilerParams(
            dimension_semantics=("parallel",),
            vmem_limit_bytes=100 * 1024 * 1024),
    )(x, prompts, keys)


# speedup vs baseline: 1.2572x; 1.2572x over previous
import jax
import jax.numpy as jnp
from jax.experimental import pallas as pl
from jax.experimental.pallas import tpu as pltpu


BLK = 2048
POOL = 64
SEL = 8


def _body(x_ref, prompts_ref, keys_ref, out_ref, kn_ref, pm_ref):
    @pl.when(pl.program_id(0) == 0)
    def _init():
        keys = keys_ref[...]
        kn_ref[...] = keys * jax.lax.rsqrt(
            jnp.maximum(jnp.sum(keys * keys, axis=1, keepdims=True), 1e-24))
        pm_ref[...] = (jnp.mean(prompts_ref[...], axis=1)
                       * (1.0 / SEL)).astype(jnp.bfloat16)

    x = x_ref[...]
    sim = jax.lax.dot_general(
        kn_ref[...], x, (((1,), (1,)), ((), ())),
        preferred_element_type=jnp.float32)

    work = sim
    for _ in range(SEL - 1):
        m = jnp.max(work, axis=0, keepdims=True)
        work = jnp.where(work < m, work, -jnp.inf)
    t = jnp.max(work, axis=0, keepdims=True)
    sel = (sim >= t).astype(jnp.bfloat16)

    pf = jax.lax.dot_general(
        sel, pm_ref[...], (((0,), (0,)), ((), ())),
        preferred_element_type=jnp.float32)
    out_ref[...] = x + pf


@jax.jit
def kernel(x, prompts, keys):
    batch, dim = x.shape
    grid = (batch // BLK,)
    return pl.pallas_call(
        _body,
        grid=grid,
        in_specs=[
            pl.BlockSpec((BLK, dim), lambda i: (i, 0)),
            pl.BlockSpec(prompts.shape, lambda i: (0, 0, 0)),
            pl.BlockSpec(keys.shape, lambda i: (0, 0)),
        ],
        out_specs=pl.BlockSpec((BLK, dim), lambda i: (i, 0)),
        out_shape=jax.ShapeDtypeStruct((batch, dim), x.dtype),
        scratch_shapes=[
            pltpu.VMEM((POOL, dim), jnp.float32),
            pltpu.VMEM((POOL, dim), jnp.bfloat16),
        ],
        compiler_params=pltpu.CompilerParams(
            dimension_semantics=("parallel",),
            vmem_limit_bytes=100 * 1024 * 1024),
    )(x, prompts, keys)
